# baseline (speedup 1.0000x reference)
import functools

import jax
import jax.numpy as jnp
from jax import lax
from jax.experimental import pallas as pl
from jax.experimental.pallas import tpu as pltpu
from jax.experimental.pallas import tpu_sc as plsc

_THRESHOLD = 64.0
_NEG = -1e30
_LANES = 128
_ROWS = 8
_B = 128
_V = 100000

_NC = 2
_NS = 16
_SC_ROWS = 64
_QCOLS = 24960
_SCV = 4 * _QCOLS
_CH = 2560
_CHT = _QCOLS - 9 * _CH


def _row_losses(x, true):
    m1 = jnp.max(x, axis=1, keepdims=True)
    eq = x == m1
    runner = jnp.max(jnp.where(eq, _NEG, x), axis=1, keepdims=True)
    cnt = jnp.sum(eq.astype(jnp.float32), axis=1, keepdims=True)
    m2 = jnp.where(cnt > 1.0, m1, runner)
    s = jnp.sum(jnp.exp(x - m1), axis=1, keepdims=True)
    masked_max = jnp.where(true == m1, m2, m1)
    margin = true - masked_max
    lse = m1 + jnp.log(s)
    l = jnp.where(margin >= 0.0, 1.0 - margin, 1.0 - true + lse)
    return jnp.maximum(l, 0.0)


def _panel_top2_sumexp(x):
    m1 = jnp.max(x, axis=1, keepdims=True)
    eq = x == m1
    runner = jnp.max(jnp.where(eq, _NEG, x), axis=1, keepdims=True)
    cnt = jnp.sum(eq.astype(jnp.float32), axis=1, keepdims=True)
    m2 = jnp.where(cnt > 1.0, m1, runner)
    s = jnp.sum(jnp.exp(x - m1), axis=1, keepdims=True)
    return m1, m2, s


def _merge_top2(a1, a2, b1, b2):
    m1 = jnp.maximum(a1, b1)
    m2 = jnp.maximum(jnp.minimum(a1, b1), jnp.where(a1 >= b1, a2, b2))
    return m1, m2


def _tc_main_kernel(xa_ref, xb_ref, true_ref, l_ref):
    l_ref[0:_ROWS, :] = _row_losses(xa_ref[...], true_ref[0:_ROWS, :])
    l_ref[_ROWS:2 * _ROWS, :] = _row_losses(xb_ref[...],
                                            true_ref[_ROWS:2 * _ROWS, :])


def _sc_kernel(x_hbm, out_hbm, buf_a, buf_b, buf_c, stage, sem_a, sem_b,
               sem_c):
    wid = lax.axis_index("s") * _NC + lax.axis_index("c")
    g = wid // 4
    q = wid % 4
    row0 = (_B - _SC_ROWS) + g * 8
    base = q * _QCOLS

    secs = [(i * _CH, _CH, (buf_a, buf_b)[i % 2], (sem_a, sem_b)[i % 2])
            for i in range(9)]
    secs.append((9 * _CH, _CHT, buf_c, sem_c))

    def _start(sec):
        rel, w, buf, sem = secs[sec]
        return pltpu.async_copy(
            x_hbm.at[pl.ds(row0, 8), pl.ds(base + rel, w)], buf, sem)

    acc = [(jnp.full((16,), _NEG, jnp.float32),
            jnp.full((16,), _NEG, jnp.float32),
            jnp.zeros((16,), jnp.float32)) for _ in range(8)]

    handle = _start(0)
    for sec in range(10):
        _, w, buf, _ = secs[sec]
        handle.wait()
        if sec + 1 < 10:
            handle = _start(sec + 1)
        for r in range(8):
            m1v, m2v, sv = acc[r]

            def _top2(i, carry):
                a1, a2 = carry
                v = buf[r, pl.ds(i * 16, 16)]
                a2 = jnp.maximum(a2, jnp.minimum(a1, v))
                a1 = jnp.maximum(a1, v)
                return (a1, a2)

            m1n, m2n = lax.fori_loop(0, w // 16, _top2, (m1v, m2v))
            sv = sv * jnp.exp(m1v - m1n)

            def _esum(i, s_acc):
                v = buf[r, pl.ds(i * 16, 16)]
                return s_acc + jnp.exp(v - m1n)

            sv = lax.fori_loop(0, w // 16, _esum, sv)
            acc[r] = (m1n, m2n, sv)

    iv = lax.iota(jnp.int32, 16)
    for r in range(8):
        m1v, m2v, sv = acc[r]
        g1 = jnp.max(m1v)
        eq = m1v == g1
        cnt = plsc.all_reduce_population_count(eq)
        runner = jnp.max(jnp.where(eq, _NEG, m1v))
        g2v = jnp.where(cnt > 1, g1, jnp.maximum(runner, jnp.max(m2v)))
        sg = jnp.sum(sv * jnp.exp(m1v - g1))
        ov = jnp.where(iv == 0, g1, jnp.where(iv == 1, g2v,
                                              jnp.where(iv == 2, sg, 0.0)))
        stage[r, :] = ov
    pltpu.sync_copy(stage, out_hbm.at[q, pl.ds(g * 8, 8), :])


def _tc_tail_kernel(l_lo_ref, scp_ref, strip_ref, true_hi_ref, res_ref):
    m1, m2, s = _panel_top2_sumexp(strip_ref[...])
    for p in range(4):
        p1 = scp_ref[p, :, 0:1]
        p2 = scp_ref[p, :, 1:2]
        ps = scp_ref[p, :, 2:3]
        n1, n2 = _merge_top2(m1, m2, p1, p2)
        s = s * jnp.exp(m1 - n1) + ps * jnp.exp(p1 - n1)
        m1, m2 = n1, n2

    true = true_hi_ref[...]
    masked_max = jnp.where(true == m1, m2, m1)
    margin = true - masked_max
    lse = m1 + jnp.log(s)
    l_hi = jnp.where(margin >= 0.0, 1.0 - margin, 1.0 - true + lse)
    l_hi = jnp.maximum(l_hi, 0.0)
    l = jnp.concatenate([l_lo_ref[...], l_hi], axis=0)

    ones_row = jnp.ones((1, _LANES), jnp.float32)
    bc = jax.lax.dot_general(l, ones_row, (((1,), (0,)), ((), ())),
                             precision=jax.lax.Precision.HIGHEST)
    br = bc.T
    ii = jax.lax.broadcasted_iota(jnp.int32, (_LANES, _LANES), 0)
    jj = jax.lax.broadcasted_iota(jnp.int32, (_LANES, _LANES), 1)
    prec = ((br < bc) | ((br == bc) & (jj < ii))).astype(jnp.float32)
    incl = jnp.where((br == bc) & (jj == ii), 1.0, prec)
    ones_col = jnp.ones((_LANES, 1), jnp.float32)
    rank = jax.lax.dot_general(prec, ones_col, (((1,), (0,)), ((), ())),
                               precision=jax.lax.Precision.HIGHEST)
    csum = jax.lax.dot_general(incl, l, (((1,), (0,)), ((), ())),
                               precision=jax.lax.Precision.HIGHEST)
    keep = (csum <= _THRESHOLD + 1.0 - rank).astype(jnp.float32)
    c1 = jnp.sum(keep * l)
    c2 = jnp.float32(_LANES) - jnp.sum(keep)
    res_ref[0, 0] = jnp.where(c1 < c2, c2, c1)


@jax.jit
def kernel(output, target):
    B, V = output.shape
    tc_rows = B - _SC_ROWS
    nsteps = tc_rows // (2 * _ROWS)
    rows = jnp.arange(B, dtype=jnp.int32)
    true = output[rows, target.astype(jnp.int32)].reshape(B, 1)

    l_lo = pl.pallas_call(
        _tc_main_kernel,
        grid=(nsteps,),
        in_specs=[
            pl.BlockSpec((_ROWS, V), lambda p: (2 * p, 0)),
            pl.BlockSpec((_ROWS, V), lambda p: (2 * p + 1, 0)),
            pl.BlockSpec((2 * _ROWS, 1), lambda p: (p, 0)),
        ],
        out_specs=pl.BlockSpec((2 * _ROWS, 1), lambda p: (p, 0)),
        out_shape=jax.ShapeDtypeStruct((tc_rows, 1), jnp.float32),
    )(output, output, true[0:tc_rows])

    sc_fn = pl.kernel(
        _sc_kernel,
        out_type=jax.ShapeDtypeStruct((4, _SC_ROWS, 16), jnp.float32),
        mesh=plsc.VectorSubcoreMesh(core_axis_name="c", subcore_axis_name="s",
                                    num_cores=_NC, num_subcores=_NS),
        scratch_types=[
            pltpu.VMEM((8, _CH), jnp.float32),
            pltpu.VMEM((8, _CH), jnp.float32),
            pltpu.VMEM((8, _CHT), jnp.float32),
            pltpu.VMEM((8, 16), jnp.float32),
            pltpu.SemaphoreType.DMA,
            pltpu.SemaphoreType.DMA,
            pltpu.SemaphoreType.DMA,
        ],
        compiler_params=pltpu.CompilerParams(needs_layout_passes=False),
    )
    scp = sc_fn(output)

    strip = lax.slice(output, (tc_rows, _SCV), (B, V))

    res = pl.pallas_call(
        _tc_tail_kernel,
        grid=(1,),
        in_specs=[
            pl.BlockSpec((tc_rows, 1), lambda i: (0, 0)),
            pl.BlockSpec((4, _SC_ROWS, 16), lambda i: (0, 0, 0)),
            pl.BlockSpec((_SC_ROWS, V - _SCV), lambda i: (0, 0)),
            pl.BlockSpec((_SC_ROWS, 1), lambda i: (0, 0)),
        ],
        out_specs=pl.BlockSpec((1, 1), lambda i: (0, 0),
                               memory_space=pltpu.SMEM),
        out_shape=jax.ShapeDtypeStruct((1, 1), jnp.float32),
    )(l_lo, scp, strip, true[tc_rows:B])
    return res[0, 0]


# speedup vs baseline: 1.7799x; 1.7799x over previous
import functools

import jax
import jax.numpy as jnp
from jax import lax
from jax.experimental import pallas as pl
from jax.experimental.pallas import tpu as pltpu
from jax.experimental.pallas import tpu_sc as plsc

_THRESHOLD = 64.0
_NEG = -1e30
_LANES = 128
_ROWS = 8
_B = 128
_V = 100000

_NC = 2
_NS = 16
_SC_ROWS = 64
_QCOLS = 24960
_SCV = 4 * _QCOLS
_CH = 4992
_NSEC = _QCOLS // _CH


def _row_losses(x, true):
    m1 = jnp.max(x, axis=1, keepdims=True)
    eq = x == m1
    runner = jnp.max(jnp.where(eq, _NEG, x), axis=1, keepdims=True)
    cnt = jnp.sum(eq.astype(jnp.float32), axis=1, keepdims=True)
    m2 = jnp.where(cnt > 1.0, m1, runner)
    s = jnp.sum(jnp.exp(x - m1), axis=1, keepdims=True)
    masked_max = jnp.where(true == m1, m2, m1)
    margin = true - masked_max
    lse = m1 + jnp.log(s)
    l = jnp.where(margin >= 0.0, 1.0 - margin, 1.0 - true + lse)
    return jnp.maximum(l, 0.0)


def _panel_top2_sumexp(x):
    m1 = jnp.max(x, axis=1, keepdims=True)
    eq = x == m1
    runner = jnp.max(jnp.where(eq, _NEG, x), axis=1, keepdims=True)
    cnt = jnp.sum(eq.astype(jnp.float32), axis=1, keepdims=True)
    m2 = jnp.where(cnt > 1.0, m1, runner)
    s = jnp.sum(jnp.exp(x - m1), axis=1, keepdims=True)
    return m1, m2, s


def _merge_top2(a1, a2, b1, b2):
    m1 = jnp.maximum(a1, b1)
    m2 = jnp.maximum(jnp.minimum(a1, b1), jnp.where(a1 >= b1, a2, b2))
    return m1, m2


def _tc_main_kernel(xa_ref, xb_ref, true_ref, l_ref):
    l_ref[0:_ROWS, :] = _row_losses(xa_ref[...], true_ref[0:_ROWS, :])
    l_ref[_ROWS:2 * _ROWS, :] = _row_losses(xb_ref[...],
                                            true_ref[_ROWS:2 * _ROWS, :])


def _sc_kernel(x_hbm, out_hbm, buf_a, buf_b, m1s, m2s, ss, stage, sem_a,
               sem_b):
    wid = lax.axis_index("s") * _NC + lax.axis_index("c")
    g = wid // 4
    q = wid % 4
    row0 = (_B - _SC_ROWS) + g * 8
    base = q * _QCOLS
    bufs = (buf_a, buf_b)
    sems = (sem_a, sem_b)
    negv = jnp.full((16,), _NEG, jnp.float32)

    def _start(sec):
        return pltpu.async_copy(
            x_hbm.at[pl.ds(row0, 8), pl.ds(base + sec * _CH, _CH)],
            bufs[sec % 2], sems[sec % 2])

    def _init(r, _):
        m1s[r, :] = negv
        m2s[r, :] = negv
        ss[r, :] = jnp.zeros((16,), jnp.float32)
        return 0

    lax.fori_loop(0, 8, _init, 0)

    handle = _start(0)
    for sec in range(_NSEC):
        handle.wait()
        if sec + 1 < _NSEC:
            handle = _start(sec + 1)
        buf = bufs[sec % 2]

        def _row(r, _):
            def _top2(i, c):
                a10, a20, a11, a21, a12, a22, a13, a23 = c
                o = i * 128
                v0 = buf[r, pl.ds(o, 16)]
                v1 = buf[r, pl.ds(o + 16, 16)]
                v2 = buf[r, pl.ds(o + 32, 16)]
                v3 = buf[r, pl.ds(o + 48, 16)]
                v4 = buf[r, pl.ds(o + 64, 16)]
                v5 = buf[r, pl.ds(o + 80, 16)]
                v6 = buf[r, pl.ds(o + 96, 16)]
                v7 = buf[r, pl.ds(o + 112, 16)]
                a20 = jnp.maximum(a20, jnp.minimum(a10, v0))
                a10 = jnp.maximum(a10, v0)
                a21 = jnp.maximum(a21, jnp.minimum(a11, v1))
                a11 = jnp.maximum(a11, v1)
                a22 = jnp.maximum(a22, jnp.minimum(a12, v2))
                a12 = jnp.maximum(a12, v2)
                a23 = jnp.maximum(a23, jnp.minimum(a13, v3))
                a13 = jnp.maximum(a13, v3)
                a20 = jnp.maximum(a20, jnp.minimum(a10, v4))
                a10 = jnp.maximum(a10, v4)
                a21 = jnp.maximum(a21, jnp.minimum(a11, v5))
                a11 = jnp.maximum(a11, v5)
                a22 = jnp.maximum(a22, jnp.minimum(a12, v6))
                a12 = jnp.maximum(a12, v6)
                a23 = jnp.maximum(a23, jnp.minimum(a13, v7))
                a13 = jnp.maximum(a13, v7)
                return (a10, a20, a11, a21, a12, a22, a13, a23)

            cpairs = lax.fori_loop(0, _CH // 128, _top2, (negv,) * 8)
            b1, b2 = cpairs[0], cpairs[1]
            for u in range(1, 4):
                b1, b2 = _merge_top2(b1, b2, cpairs[2 * u], cpairs[2 * u + 1])
            m1v = m1s[r, :]
            m1n, m2n = _merge_top2(m1v, m2s[r, :], b1, b2)
            m1s[r, :] = m1n
            m2s[r, :] = m2n
            sv = ss[r, :] * jnp.exp(m1v - m1n)

            def _esum(i, c):
                s0, s1, s2, s3 = c
                o = i * 128
                s0 = s0 + jnp.exp(buf[r, pl.ds(o, 16)] - m1n)
                s1 = s1 + jnp.exp(buf[r, pl.ds(o + 16, 16)] - m1n)
                s2 = s2 + jnp.exp(buf[r, pl.ds(o + 32, 16)] - m1n)
                s3 = s3 + jnp.exp(buf[r, pl.ds(o + 48, 16)] - m1n)
                s0 = s0 + jnp.exp(buf[r, pl.ds(o + 64, 16)] - m1n)
                s1 = s1 + jnp.exp(buf[r, pl.ds(o + 80, 16)] - m1n)
                s2 = s2 + jnp.exp(buf[r, pl.ds(o + 96, 16)] - m1n)
                s3 = s3 + jnp.exp(buf[r, pl.ds(o + 112, 16)] - m1n)
                return (s0, s1, s2, s3)

            zs = jnp.zeros((16,), jnp.float32)
            s0, s1, s2, s3 = lax.fori_loop(0, _CH // 128, _esum,
                                           (zs, zs, zs, zs))
            ss[r, :] = sv + (s0 + s1) + (s2 + s3)
            return 0

        lax.fori_loop(0, 8, _row, 0)

    iv = lax.iota(jnp.int32, 16)

    def _fold(r, _):
        m1v = m1s[r, :]
        m2v = m2s[r, :]
        sv = ss[r, :]
        g1 = jnp.max(m1v)
        eq = m1v == g1
        cnt = plsc.all_reduce_population_count(eq)
        runner = jnp.max(jnp.where(eq, _NEG, m1v))
        g2v = jnp.where(cnt > 1, g1, jnp.maximum(runner, jnp.max(m2v)))
        sg = jnp.sum(sv * jnp.exp(m1v - g1))
        ov = jnp.where(iv == 0, g1, jnp.where(iv == 1, g2v,
                                              jnp.where(iv == 2, sg, 0.0)))
        stage[r, :] = ov
        return 0

    lax.fori_loop(0, 8, _fold, 0)
    pltpu.sync_copy(stage, out_hbm.at[q, pl.ds(g * 8, 8), :])


def _tc_tail_kernel(l_lo_ref, scp_ref, strip_ref, true_hi_ref, res_ref):
    m1, m2, s = _panel_top2_sumexp(strip_ref[...])
    for p in range(4):
        p1 = scp_ref[p, :, 0:1]
        p2 = scp_ref[p, :, 1:2]
        ps = scp_ref[p, :, 2:3]
        n1, n2 = _merge_top2(m1, m2, p1, p2)
        s = s * jnp.exp(m1 - n1) + ps * jnp.exp(p1 - n1)
        m1, m2 = n1, n2

    true = true_hi_ref[...]
    masked_max = jnp.where(true == m1, m2, m1)
    margin = true - masked_max
    lse = m1 + jnp.log(s)
    l_hi = jnp.where(margin >= 0.0, 1.0 - margin, 1.0 - true + lse)
    l_hi = jnp.maximum(l_hi, 0.0)
    l = jnp.concatenate([l_lo_ref[...], l_hi], axis=0)

    ones_row = jnp.ones((1, _LANES), jnp.float32)
    bc = jax.lax.dot_general(l, ones_row, (((1,), (0,)), ((), ())),
                             precision=jax.lax.Precision.HIGHEST)
    br = bc.T
    ii = jax.lax.broadcasted_iota(jnp.int32, (_LANES, _LANES), 0)
    jj = jax.lax.broadcasted_iota(jnp.int32, (_LANES, _LANES), 1)
    prec = ((br < bc) | ((br == bc) & (jj < ii))).astype(jnp.float32)
    incl = jnp.where((br == bc) & (jj == ii), 1.0, prec)
    ones_col = jnp.ones((_LANES, 1), jnp.float32)
    rank = jax.lax.dot_general(prec, ones_col, (((1,), (0,)), ((), ())),
                               precision=jax.lax.Precision.HIGHEST)
    csum = jax.lax.dot_general(incl, l, (((1,), (0,)), ((), ())),
                               precision=jax.lax.Precision.HIGHEST)
    keep = (csum <= _THRESHOLD + 1.0 - rank).astype(jnp.float32)
    c1 = jnp.sum(keep * l)
    c2 = jnp.float32(_LANES) - jnp.sum(keep)
    res_ref[0, 0] = jnp.where(c1 < c2, c2, c1)


@jax.jit
def kernel(output, target):
    B, V = output.shape
    tc_rows = B - _SC_ROWS
    nsteps = tc_rows // (2 * _ROWS)
    rows = jnp.arange(B, dtype=jnp.int32)
    true = output[rows, target.astype(jnp.int32)].reshape(B, 1)

    l_lo = pl.pallas_call(
        _tc_main_kernel,
        grid=(nsteps,),
        in_specs=[
            pl.BlockSpec((_ROWS, V), lambda p: (2 * p, 0)),
            pl.BlockSpec((_ROWS, V), lambda p: (2 * p + 1, 0)),
            pl.BlockSpec((2 * _ROWS, 1), lambda p: (p, 0)),
        ],
        out_specs=pl.BlockSpec((2 * _ROWS, 1), lambda p: (p, 0)),
        out_shape=jax.ShapeDtypeStruct((tc_rows, 1), jnp.float32),
    )(output, output, true[0:tc_rows])

    sc_fn = pl.kernel(
        _sc_kernel,
        out_type=jax.ShapeDtypeStruct((4, _SC_ROWS, 16), jnp.float32),
        mesh=plsc.VectorSubcoreMesh(core_axis_name="c", subcore_axis_name="s",
                                    num_cores=_NC, num_subcores=_NS),
        scratch_types=[
            pltpu.VMEM((8, _CH), jnp.float32),
            pltpu.VMEM((8, _CH), jnp.float32),
            pltpu.VMEM((8, 16), jnp.float32),
            pltpu.VMEM((8, 16), jnp.float32),
            pltpu.VMEM((8, 16), jnp.float32),
            pltpu.VMEM((8, 16), jnp.float32),
            pltpu.SemaphoreType.DMA,
            pltpu.SemaphoreType.DMA,
        ],
        compiler_params=pltpu.CompilerParams(needs_layout_passes=False),
    )
    scp = sc_fn(output)

    strip = lax.slice(output, (tc_rows, _SCV), (B, V))

    res = pl.pallas_call(
        _tc_tail_kernel,
        grid=(1,),
        in_specs=[
            pl.BlockSpec((tc_rows, 1), lambda i: (0, 0)),
            pl.BlockSpec((4, _SC_ROWS, 16), lambda i: (0, 0, 0)),
            pl.BlockSpec((_SC_ROWS, V - _SCV), lambda i: (0, 0)),
            pl.BlockSpec((_SC_ROWS, 1), lambda i: (0, 0)),
        ],
        out_specs=pl.BlockSpec((1, 1), lambda i: (0, 0),
                               memory_space=pltpu.SMEM),
        out_shape=jax.ShapeDtypeStruct((1, 1), jnp.float32),
    )(l_lo, scp, strip, true[tc_rows:B])
    return res[0, 0]
